# baseline (speedup 1.0000x reference)
import jax
import jax.numpy as jnp
from jax.experimental import pallas as pl


def _pe_add_kernel(x_ref, pe_ref, o_ref):
    o_ref[...] = x_ref[...] + pe_ref[...]


def kernel(x, pe_table):
    B, L, D = x.shape
    TL = 256
    nl = L // TL
    x2 = x.reshape(B * L, D)
    out = pl.pallas_call(
        _pe_add_kernel,
        grid=(nl, B),
        in_specs=[
            pl.BlockSpec((TL, D), lambda i, j: (j * nl + i, 0)),
            pl.BlockSpec((TL, D), lambda i, j: (i, 0)),
        ],
        out_specs=pl.BlockSpec((TL, D), lambda i, j: (j * nl + i, 0)),
        out_shape=jax.ShapeDtypeStruct((B * L, D), x.dtype),
    )(x2, pe_table[:L])
    return out.reshape(B, L, D)


# speedup vs baseline: 1.1406x; 1.1406x over previous
import jax
import jax.numpy as jnp
from jax.experimental import pallas as pl


def _pe_add_kernel(x_ref, pe_ref, o_ref):
    o_ref[...] = x_ref[...] + pe_ref[...]


def kernel(x, pe_table):
    B, L, D = x.shape
    TL = 128
    return pl.pallas_call(
        _pe_add_kernel,
        grid=(L // TL,),
        in_specs=[
            pl.BlockSpec((B, TL, D), lambda i: (0, i, 0)),
            pl.BlockSpec((TL, D), lambda i: (i, 0)),
        ],
        out_specs=pl.BlockSpec((B, TL, D), lambda i: (0, i, 0)),
        out_shape=jax.ShapeDtypeStruct((B, L, D), x.dtype),
    )(x, pe_table[:L])


# speedup vs baseline: 1.1530x; 1.0109x over previous
import jax
import jax.numpy as jnp
from jax.experimental import pallas as pl


def _pe_add_kernel(x_ref, pe_ref, o_ref):
    o_ref[...] = x_ref[...] + pe_ref[...]


def kernel(x, pe_table):
    B, L, D = x.shape
    TL = 256
    return pl.pallas_call(
        _pe_add_kernel,
        grid=(L // TL,),
        in_specs=[
            pl.BlockSpec((B, TL, D), lambda i: (0, i, 0)),
            pl.BlockSpec((TL, D), lambda i: (i, 0)),
        ],
        out_specs=pl.BlockSpec((B, TL, D), lambda i: (0, i, 0)),
        out_shape=jax.ShapeDtypeStruct((B, L, D), x.dtype),
    )(x, pe_table[:L])
